# Initial kernel scaffold; baseline (speedup 1.0000x reference)
import jax
import jax.numpy as jnp
from jax.experimental import pallas as pl


def kernel(x_omic, emb_var, emb_vc, emb_func, W, b):
    raise NotImplementedError("write your pallas kernel here")


# speedup vs baseline: 5.5618x; 5.5618x over previous
import functools

import jax
import jax.numpy as jnp
from jax import lax
from jax.experimental import pallas as pl
from jax.experimental.pallas import tpu as pltpu
from jax.experimental.pallas import tpu_sc as plsc

_NC = 2
_NS = 16
_NW = _NC * _NS
_CHUNK = 128

_D_VAR = 128
_OUT = 256


def _sc_gather(table, idx, n_pad):
    k_chunks = n_pad // (_NW * _CHUNK)
    per_w = k_chunks * _CHUNK
    mesh = plsc.VectorSubcoreMesh(core_axis_name="c", subcore_axis_name="s")

    @functools.partial(
        pl.kernel,
        mesh=mesh,
        out_type=jax.ShapeDtypeStruct((n_pad, _D_VAR), jnp.float32),
        scratch_types=[
            pltpu.VMEM((_CHUNK,), jnp.int32),
            pltpu.VMEM((_CHUNK, _D_VAR), jnp.float32),
            pltpu.SemaphoreType.DMA,
        ],
    )
    def gather_kernel(table_hbm, idx_hbm, out_hbm, idx_v, rows_v, sem):
        wid = lax.axis_index("s") * _NC + lax.axis_index("c")
        base = wid * per_w

        def body(j, carry):
            off = base + j * _CHUNK
            pltpu.sync_copy(idx_hbm.at[pl.ds(off, _CHUNK)], idx_v)
            pltpu.async_copy(table_hbm.at[idx_v], rows_v, sem).wait()
            pltpu.sync_copy(rows_v, out_hbm.at[pl.ds(off, _CHUNK)])
            return carry

        lax.fori_loop(0, k_chunks, body, 0)

    return gather_kernel(table, idx)


def _tc_body(x_ref, g_ref, vc_ref, f_ref, w_ref, b_ref, o_ref):
    x = x_ref[...]
    g = g_ref[...]
    acc = jnp.dot(g, w_ref[0:128, :], preferred_element_type=jnp.float32)

    vc_iota = lax.broadcasted_iota(jnp.int32, (1, 33), 1)
    oh_vc = (x[:, 1:2].astype(jnp.int32) == vc_iota).astype(jnp.float32)
    h_vc = jnp.dot(oh_vc, vc_ref[...], preferred_element_type=jnp.float32)
    acc += jnp.dot(h_vc, w_ref[128:160, :], preferred_element_type=jnp.float32)

    f_iota = lax.broadcasted_iota(jnp.int32, (1, 65), 1)
    counts = (x[:, 2:3].astype(jnp.int32) == f_iota).astype(jnp.float32)
    for j in range(3, 8):
        counts += (x[:, j:j + 1].astype(jnp.int32) == f_iota).astype(jnp.float32)
    h_f = jnp.dot(counts, f_ref[...], preferred_element_type=jnp.float32)
    h_f = h_f * (1.0 / 6.0)
    acc += jnp.dot(h_f, w_ref[160:192, :], preferred_element_type=jnp.float32)

    acc += x[:, 8:9] * w_ref[192:193, :]
    acc += b_ref[...]

    o_ref[...] = jnp.where(acc > 0.0, acc, jnp.exp(acc) - 1.0)


def kernel(x_omic, emb_var, emb_vc, emb_func, W, b):
    B, L, _ = x_omic.shape
    n = B * L
    x2 = x_omic.reshape(n, 9)

    T = 960
    n_pad = -(-n // (_NW * _CHUNK)) * (_NW * _CHUNK)
    var_idx = x2[:, 0].astype(jnp.int32)
    var_idx = jnp.pad(var_idx, (0, n_pad - n))

    gathered = _sc_gather(emb_var, var_idx, n_pad)

    grid = n // T
    out2 = pl.pallas_call(
        _tc_body,
        grid=(grid,),
        in_specs=[
            pl.BlockSpec((T, 9), lambda i: (i, 0)),
            pl.BlockSpec((T, _D_VAR), lambda i: (i, 0)),
            pl.BlockSpec((33, 32), lambda i: (0, 0)),
            pl.BlockSpec((65, 32), lambda i: (0, 0)),
            pl.BlockSpec((193, _OUT), lambda i: (0, 0)),
            pl.BlockSpec((1, _OUT), lambda i: (0, 0)),
        ],
        out_specs=pl.BlockSpec((T, _OUT), lambda i: (i, 0)),
        out_shape=jax.ShapeDtypeStruct((n, _OUT), jnp.float32),
    )(x2, gathered, emb_vc, emb_func, W, b.reshape(1, _OUT))

    return out2.reshape(B, L, _OUT)


# speedup vs baseline: 7.5313x; 1.3541x over previous
import functools

import jax
import jax.numpy as jnp
from jax import lax
from jax.experimental import pallas as pl
from jax.experimental.pallas import tpu as pltpu
from jax.experimental.pallas import tpu_sc as plsc

_NC = 2
_NS = 16
_NW = _NC * _NS
_CHUNK = 128
_NBUF = 3

_D_VAR = 128
_OUT = 256


def _sc_gather(table, idx, n_pad):
    per_w = n_pad // _NW
    k_chunks = per_w // _CHUNK
    assert per_w % _CHUNK == 0 and k_chunks % _NBUF == 0
    mesh = plsc.VectorSubcoreMesh(core_axis_name="c", subcore_axis_name="s")

    @functools.partial(
        pl.kernel,
        mesh=mesh,
        out_type=jax.ShapeDtypeStruct((n_pad, _D_VAR), jnp.float32),
        scratch_types=[
            pltpu.VMEM((per_w,), jnp.int32),
        ] + [pltpu.VMEM((_CHUNK, _D_VAR), jnp.float32)] * _NBUF
          + [pltpu.SemaphoreType.DMA] * (2 * _NBUF),
    )
    def gather_kernel(table_hbm, idx_hbm, out_hbm, idxall, r0, r1, r2,
                      g0, g1, g2, w0, w1, w2):
        rows = [r0, r1, r2]
        gsem = [g0, g1, g2]
        wsem = [w0, w1, w2]
        wid = lax.axis_index("s") * _NC + lax.axis_index("c")
        base = wid * per_w

        pltpu.sync_copy(idx_hbm.at[pl.ds(base, per_w)], idxall)

        def fire_gather(j, b):
            pltpu.async_copy(
                table_hbm.at[idxall.at[pl.ds(j * _CHUNK, _CHUNK)]],
                rows[b], gsem[b])

        def fire_wb(j, b):
            pltpu.async_copy(
                rows[b], out_hbm.at[pl.ds(base + j * _CHUNK, _CHUNK)],
                wsem[b])

        def wait_g(b):
            pltpu.make_async_copy(
                table_hbm.at[idxall.at[pl.ds(0, _CHUNK)]], rows[b],
                gsem[b]).wait()

        def wait_w(b):
            pltpu.make_async_copy(
                rows[b], out_hbm.at[pl.ds(base, _CHUNK)], wsem[b]).wait()

        def body(g, carry):
            for b in range(_NBUF):
                jb = _NBUF * g + b
                pb = (b + _NBUF - 1) % _NBUF

                @pl.when(g > 0)
                def _():
                    wait_w(b)

                fire_gather(jb, b)

                if b == 0:
                    @pl.when(g > 0)
                    def _():
                        wait_g(pb)
                        fire_wb(_NBUF * g - 1, pb)
                else:
                    wait_g(pb)
                    fire_wb(jb - 1, pb)
            return carry

        lax.fori_loop(0, k_chunks // _NBUF, body, 0)
        wait_g(_NBUF - 1)
        fire_wb(k_chunks - 1, _NBUF - 1)
        for b in range(_NBUF):
            wait_w(b)

    return gather_kernel(table, idx)


def _tc_body(x_ref, g_ref, vc_ref, f_ref, w_ref, b_ref, o_ref):
    x = x_ref[0]
    g = g_ref[0, 0:1425, :]
    acc = jnp.dot(g, w_ref[0:128, :], preferred_element_type=jnp.float32)

    vc_iota = lax.broadcasted_iota(jnp.int32, (1, 33), 1)
    oh_vc = (x[:, 1:2].astype(jnp.int32) == vc_iota).astype(jnp.float32)
    h_vc = jnp.dot(oh_vc, vc_ref[...], preferred_element_type=jnp.float32)
    acc += jnp.dot(h_vc, w_ref[128:160, :], preferred_element_type=jnp.float32)

    f_iota = lax.broadcasted_iota(jnp.int32, (1, 65), 1)
    counts = (x[:, 2:3].astype(jnp.int32) == f_iota).astype(jnp.float32)
    for j in range(3, 8):
        counts += (x[:, j:j + 1].astype(jnp.int32) == f_iota).astype(jnp.float32)
    h_f = jnp.dot(counts, f_ref[...], preferred_element_type=jnp.float32)
    h_f = h_f * (1.0 / 6.0)
    acc += jnp.dot(h_f, w_ref[160:192, :], preferred_element_type=jnp.float32)

    acc += x[:, 8:9] * w_ref[192:193, :]
    acc += b_ref[...]

    o_ref[0] = jnp.where(acc > 0.0, acc, jnp.exp(acc) - 1.0)


def kernel(x_omic, emb_var, emb_vc, emb_func, W, b):
    B, L, _ = x_omic.shape
    step = _NW * _CHUNK * _NBUF // B if (_NW * _CHUNK * _NBUF) % B == 0 else _NW * _CHUNK * _NBUF
    l_pad = -(-L // step) * step
    n_pad = B * l_pad

    var_ids = x_omic[..., 0].astype(jnp.int32)
    var_ids = jnp.pad(var_ids, ((0, 0), (0, l_pad - L)))
    gathered = _sc_gather(emb_var, var_ids.reshape(n_pad), n_pad)
    g3 = gathered.reshape(B, l_pad, _D_VAR)

    out = pl.pallas_call(
        _tc_body,
        grid=(B,),
        in_specs=[
            pl.BlockSpec((1, L, 9), lambda i: (i, 0, 0)),
            pl.BlockSpec((1, l_pad, _D_VAR), lambda i: (i, 0, 0)),
            pl.BlockSpec((33, 32), lambda i: (0, 0)),
            pl.BlockSpec((65, 32), lambda i: (0, 0)),
            pl.BlockSpec((193, _OUT), lambda i: (0, 0)),
            pl.BlockSpec((1, _OUT), lambda i: (0, 0)),
        ],
        out_specs=pl.BlockSpec((1, L, _OUT), lambda i: (i, 0, 0)),
        out_shape=jax.ShapeDtypeStruct((B, L, _OUT), jnp.float32),
    )(x_omic, g3, emb_vc, emb_func, W, b.reshape(1, _OUT))

    return out


# speedup vs baseline: 8.0708x; 1.0716x over previous
import functools

import jax
import jax.numpy as jnp
from jax import lax
from jax.experimental import pallas as pl
from jax.experimental.pallas import tpu as pltpu
from jax.experimental.pallas import tpu_sc as plsc

_NC = 2
_NS = 16
_NW = _NC * _NS
_CHUNK = 128
_NBUF = 3

_D_VAR = 128
_OUT = 256


def _sc_gather(table, idx, n_pad):
    per_w = n_pad // _NW
    k_chunks = per_w // _CHUNK
    assert per_w % _CHUNK == 0 and k_chunks % _NBUF == 0
    mesh = plsc.VectorSubcoreMesh(core_axis_name="c", subcore_axis_name="s")

    @functools.partial(
        pl.kernel,
        mesh=mesh,
        out_type=jax.ShapeDtypeStruct((n_pad, _D_VAR), jnp.float32),
        scratch_types=[
            pltpu.VMEM((per_w,), jnp.int32),
        ] + [pltpu.VMEM((_CHUNK, _D_VAR), jnp.float32)] * _NBUF
          + [pltpu.SemaphoreType.DMA] * (2 * _NBUF),
    )
    def gather_kernel(table_hbm, idx_hbm, out_hbm, idxall, r0, r1, r2,
                      g0, g1, g2, w0, w1, w2):
        rows = [r0, r1, r2]
        gsem = [g0, g1, g2]
        wsem = [w0, w1, w2]
        wid = lax.axis_index("s") * _NC + lax.axis_index("c")
        base = wid * per_w

        pltpu.sync_copy(idx_hbm.at[pl.ds(base, per_w)], idxall)

        def fire_gather(j, b):
            pltpu.async_copy(
                table_hbm.at[idxall.at[pl.ds(j * _CHUNK, _CHUNK)]],
                rows[b], gsem[b])

        def fire_wb(j, b):
            pltpu.async_copy(
                rows[b], out_hbm.at[pl.ds(base + j * _CHUNK, _CHUNK)],
                wsem[b])

        def wait_g(b):
            pltpu.make_async_copy(
                table_hbm.at[idxall.at[pl.ds(0, _CHUNK)]], rows[b],
                gsem[b]).wait()

        def wait_w(b):
            pltpu.make_async_copy(
                rows[b], out_hbm.at[pl.ds(base, _CHUNK)], wsem[b]).wait()

        def body(g, carry):
            for b in range(_NBUF):
                jb = _NBUF * g + b
                pb = (b + _NBUF - 1) % _NBUF

                @pl.when(g > 0)
                def _():
                    wait_w(b)

                fire_gather(jb, b)

                if b == 0:
                    @pl.when(g > 0)
                    def _():
                        wait_g(pb)
                        fire_wb(_NBUF * g - 1, pb)
                else:
                    wait_g(pb)
                    fire_wb(jb - 1, pb)
            return carry

        lax.fori_loop(0, k_chunks // _NBUF, body, 0)
        wait_g(_NBUF - 1)
        fire_wb(k_chunks - 1, _NBUF - 1)
        for b in range(_NBUF):
            wait_w(b)

    return gather_kernel(table, idx)


def _dot_t(lhs, rhs):
    return lax.dot_general(lhs, rhs, (((0,), (0,)), ((), ())),
                           preferred_element_type=jnp.float32)


def _tc_body(g_ref, c1_ref, c2_ref, vaf_ref, vc_ref, f_ref, w_ref, b_ref, o_ref):
    L = o_ref.shape[1]
    g = g_ref[0, 0:L, :]
    acc = jnp.dot(g, w_ref[0:128, :], preferred_element_type=jnp.float32)

    c1 = c1_ref[0]
    c2 = c2_ref[0]

    vc_iota = lax.broadcasted_iota(jnp.int32, (33, L), 0)
    oh_vc = ((c1 & 63) == vc_iota).astype(jnp.float32)
    wvc = jnp.dot(vc_ref[...], w_ref[128:160, :],
                  preferred_element_type=jnp.float32)
    acc += _dot_t(oh_vc, wvc)

    f_iota = lax.broadcasted_iota(jnp.int32, (65, L), 0)
    counts = (((c1 >> 6) & 127) == f_iota).astype(jnp.float32)
    counts += (((c1 >> 13) & 127) == f_iota).astype(jnp.float32)
    counts += (((c1 >> 20) & 127) == f_iota).astype(jnp.float32)
    counts += ((c2 & 127) == f_iota).astype(jnp.float32)
    counts += (((c2 >> 7) & 127) == f_iota).astype(jnp.float32)
    counts += (((c2 >> 14) & 127) == f_iota).astype(jnp.float32)
    wf = jnp.dot(f_ref[...], w_ref[160:192, :],
                 preferred_element_type=jnp.float32) * (1.0 / 6.0)
    acc += _dot_t(counts, wf)

    acc += _dot_t(vaf_ref[0], w_ref[192:193, :])
    acc += b_ref[...]

    o_ref[0] = jnp.where(acc > 0.0, acc, jnp.exp(acc) - 1.0)


def kernel(x_omic, emb_var, emb_vc, emb_func, W, b):
    B, L, _ = x_omic.shape
    step = _NW * _CHUNK * _NBUF // B if (_NW * _CHUNK * _NBUF) % B == 0 else _NW * _CHUNK * _NBUF
    l_pad = -(-L // step) * step
    n_pad = B * l_pad

    var_ids = x_omic[..., 0].astype(jnp.int32)
    var_ids = jnp.pad(var_ids, ((0, 0), (0, l_pad - L)))
    gathered = _sc_gather(emb_var, var_ids.reshape(n_pad), n_pad)
    g3 = gathered.reshape(B, l_pad, _D_VAR)

    ids = x_omic[..., 1:8].astype(jnp.int32)
    c1 = (ids[..., 0] | (ids[..., 1] << 6) | (ids[..., 2] << 13)
          | (ids[..., 3] << 20)).reshape(B, 1, L)
    c2 = (ids[..., 4] | (ids[..., 5] << 7)
          | (ids[..., 6] << 14)).reshape(B, 1, L)
    vaf3 = x_omic[..., 8].reshape(B, 1, L)

    out = pl.pallas_call(
        _tc_body,
        grid=(B,),
        in_specs=[
            pl.BlockSpec((1, l_pad, _D_VAR), lambda i: (i, 0, 0)),
            pl.BlockSpec((1, 1, L), lambda i: (i, 0, 0)),
            pl.BlockSpec((1, 1, L), lambda i: (i, 0, 0)),
            pl.BlockSpec((1, 1, L), lambda i: (i, 0, 0)),
            pl.BlockSpec((33, 32), lambda i: (0, 0)),
            pl.BlockSpec((65, 32), lambda i: (0, 0)),
            pl.BlockSpec((193, _OUT), lambda i: (0, 0)),
            pl.BlockSpec((1, _OUT), lambda i: (0, 0)),
        ],
        out_specs=pl.BlockSpec((1, L, _OUT), lambda i: (i, 0, 0)),
        out_shape=jax.ShapeDtypeStruct((B, L, _OUT), jnp.float32),
    )(g3, c1, c2, vaf3, emb_vc, emb_func, W, b.reshape(1, _OUT))

    return out


# speedup vs baseline: 8.9896x; 1.1138x over previous
import functools

import jax
import jax.numpy as jnp
from jax import lax
from jax.experimental import pallas as pl
from jax.experimental.pallas import tpu as pltpu
from jax.experimental.pallas import tpu_sc as plsc

_NC = 2
_NS = 16
_NW = _NC * _NS
_CHUNK = 120
_NBUF = 3
_NSLICE = 4

_D_VAR = 128
_OUT = 256


def _sc_gather(table, idx_full, n_rows, slice_base):
    per_w = n_rows // _NW
    k_chunks = per_w // _CHUNK
    assert per_w % _CHUNK == 0 and k_chunks % _NBUF == 0
    mesh = plsc.VectorSubcoreMesh(core_axis_name="c", subcore_axis_name="s")

    @functools.partial(
        pl.kernel,
        mesh=mesh,
        out_type=jax.ShapeDtypeStruct((n_rows, _D_VAR), jnp.float32),
        scratch_types=[
            pltpu.VMEM((per_w,), jnp.int32),
        ] + [pltpu.VMEM((_CHUNK, _D_VAR), jnp.float32)] * _NBUF
          + [pltpu.SemaphoreType.DMA] * (2 * _NBUF),
    )
    def gather_kernel(table_hbm, idx_hbm, out_hbm, idxall, r0, r1, r2,
                      g0, g1, g2, w0, w1, w2):
        rows = [r0, r1, r2]
        gsem = [g0, g1, g2]
        wsem = [w0, w1, w2]
        wid = lax.axis_index("s") * _NC + lax.axis_index("c")
        base = wid * per_w

        pltpu.sync_copy(idx_hbm.at[pl.ds(slice_base + base, per_w)], idxall)

        def fire_gather(j, b):
            pltpu.async_copy(
                table_hbm.at[idxall.at[pl.ds(j * _CHUNK, _CHUNK)]],
                rows[b], gsem[b])

        def fire_wb(j, b):
            pltpu.async_copy(
                rows[b], out_hbm.at[pl.ds(base + j * _CHUNK, _CHUNK)],
                wsem[b])

        def wait_g(b):
            pltpu.make_async_copy(
                table_hbm.at[idxall.at[pl.ds(0, _CHUNK)]], rows[b],
                gsem[b]).wait()

        def wait_w(b):
            pltpu.make_async_copy(
                rows[b], out_hbm.at[pl.ds(base, _CHUNK)], wsem[b]).wait()

        def body(g, carry):
            for b in range(_NBUF):
                jb = _NBUF * g + b
                pb = (b + _NBUF - 1) % _NBUF

                @pl.when(g > 0)
                def _():
                    wait_w(b)

                fire_gather(jb, b)

                if b == 0:
                    @pl.when(g > 0)
                    def _():
                        wait_g(pb)
                        fire_wb(_NBUF * g - 1, pb)
                else:
                    wait_g(pb)
                    fire_wb(jb - 1, pb)
            return carry

        lax.fori_loop(0, k_chunks // _NBUF, body, 0)
        wait_g(_NBUF - 1)
        fire_wb(k_chunks - 1, _NBUF - 1)
        for b in range(_NBUF):
            wait_w(b)

    return gather_kernel(table, idx_full)


def _dot_t(lhs, rhs):
    return lax.dot_general(lhs, rhs, (((0,), (0,)), ((), ())),
                           preferred_element_type=jnp.float32)


def _tc_compute(g_ref, c1_ref, c2_ref, vaf_ref, vc_ref, f_ref, w_ref, b_ref,
                o_ref):
    L = o_ref.shape[1]
    g = g_ref[0, 0:L, :]
    acc = jnp.dot(g, w_ref[0:128, :], preferred_element_type=jnp.float32)

    c1 = c1_ref[0]
    c2 = c2_ref[0]

    vc_iota = lax.broadcasted_iota(jnp.int32, (33, L), 0)
    oh_vc = ((c1 & 63) == vc_iota).astype(jnp.float32)
    wvc = jnp.dot(vc_ref[...], w_ref[128:160, :],
                  preferred_element_type=jnp.float32)
    acc += _dot_t(oh_vc, wvc)

    f_iota = lax.broadcasted_iota(jnp.int32, (65, L), 0)
    counts = (((c1 >> 6) & 127) == f_iota).astype(jnp.float32)
    counts += (((c1 >> 13) & 127) == f_iota).astype(jnp.float32)
    counts += (((c1 >> 20) & 127) == f_iota).astype(jnp.float32)
    counts += ((c2 & 127) == f_iota).astype(jnp.float32)
    counts += (((c2 >> 7) & 127) == f_iota).astype(jnp.float32)
    counts += (((c2 >> 14) & 127) == f_iota).astype(jnp.float32)
    wf = jnp.dot(f_ref[...], w_ref[160:192, :],
                 preferred_element_type=jnp.float32) * (1.0 / 6.0)
    acc += _dot_t(counts, wf)

    acc += _dot_t(vaf_ref[0], w_ref[192:193, :])
    acc += b_ref[...]

    o_ref[0] = jnp.where(acc > 0.0, acc, jnp.exp(acc) - 1.0)


def _tc_body_first(g_ref, c1_ref, c2_ref, vaf_ref, vc_ref, f_ref, w_ref,
                   b_ref, o_ref):
    _tc_compute(g_ref, c1_ref, c2_ref, vaf_ref, vc_ref, f_ref, w_ref, b_ref,
                o_ref)


def _tc_body_acc(o_prev_ref, g_ref, c1_ref, c2_ref, vaf_ref, vc_ref, f_ref,
                 w_ref, b_ref, o_ref):
    del o_prev_ref
    _tc_compute(g_ref, c1_ref, c2_ref, vaf_ref, vc_ref, f_ref, w_ref, b_ref,
                o_ref)


def kernel(x_omic, emb_var, emb_vc, emb_func, W, b):
    B, L, _ = x_omic.shape
    step = 1440 if L <= 1440 else -(-L // 1440) * 1440
    l_pad = step
    n_pad = B * l_pad
    bs = B // _NSLICE
    n_sl = bs * l_pad

    var_ids = x_omic[..., 0].astype(jnp.int32)
    var_ids = jnp.pad(var_ids, ((0, 0), (0, l_pad - L)))
    idx_flat = var_ids.reshape(n_pad)

    ids = x_omic[..., 1:8].astype(jnp.int32)
    c1 = (ids[..., 0] | (ids[..., 1] << 6) | (ids[..., 2] << 13)
          | (ids[..., 3] << 20)).reshape(B, 1, L)
    c2 = (ids[..., 4] | (ids[..., 5] << 7)
          | (ids[..., 6] << 14)).reshape(B, 1, L)
    vaf3 = x_omic[..., 8].reshape(B, 1, L)

    b2 = b.reshape(1, _OUT)
    out_shape = jax.ShapeDtypeStruct((B, L, _OUT), jnp.float32)

    def specs(off):
        return [
            pl.BlockSpec((1, l_pad, _D_VAR), lambda i: (i, 0, 0)),
            pl.BlockSpec((1, 1, L), lambda i, o=off: (i + o, 0, 0)),
            pl.BlockSpec((1, 1, L), lambda i, o=off: (i + o, 0, 0)),
            pl.BlockSpec((1, 1, L), lambda i, o=off: (i + o, 0, 0)),
            pl.BlockSpec((33, 32), lambda i: (0, 0)),
            pl.BlockSpec((65, 32), lambda i: (0, 0)),
            pl.BlockSpec((193, _OUT), lambda i: (0, 0)),
            pl.BlockSpec((1, _OUT), lambda i: (0, 0)),
        ]

    out = None
    for s in range(_NSLICE):
        gath = _sc_gather(emb_var, idx_flat, n_sl, s * n_sl)
        g3 = gath.reshape(bs, l_pad, _D_VAR)
        off = s * bs
        out_spec = pl.BlockSpec((1, L, _OUT), lambda i, o=off: (i + o, 0, 0))
        if s == 0:
            out = pl.pallas_call(
                _tc_body_first,
                grid=(bs,),
                in_specs=specs(off),
                out_specs=out_spec,
                out_shape=out_shape,
            )(g3, c1, c2, vaf3, emb_vc, emb_func, W, b2)
        else:
            out = pl.pallas_call(
                _tc_body_acc,
                grid=(bs,),
                in_specs=[pl.BlockSpec(memory_space=pl.ANY)] + specs(off),
                out_specs=out_spec,
                out_shape=out_shape,
                input_output_aliases={0: 0},
            )(out, g3, c1, c2, vaf3, emb_vc, emb_func, W, b2)

    return out
